# baseline (speedup 1.0000x reference)
import functools

import jax
import jax.numpy as jnp
import numpy as np
from jax.experimental import pallas as pl
from jax.experimental.pallas import tpu as pltpu
from jax.sharding import Mesh, PartitionSpec as P

_THRESHOLD = 0.5


def _gate_kernel(x_ref, wa_ref, m_ref, scale_ref, bias_ref,
                 gout_ref, gw_ref, xb_ref):
    xb = x_ref[...].astype(jnp.bfloat16)
    xb_ref[...] = xb
    gh = jax.lax.dot_general(
        xb, wa_ref[...].astype(jnp.bfloat16), (((1,), (1,)), ((), ())),
        preferred_element_type=jnp.float32)
    g2 = gh * gh
    s2 = jax.lax.dot_general(
        g2, m_ref[...], (((1,), (0,)), ((), ())),
        precision=jax.lax.Precision.HIGHEST,
        preferred_element_type=jnp.float32)
    scores = jnp.sqrt(s2 + 1e-6) * scale_ref[...] - bias_ref[...]
    mask = scores >= _THRESHOLD
    gout_ref[...] = jnp.where(mask, scores, -jnp.inf)
    gw_ref[...] = jnp.where(mask, scores, 0.0)


def _ffn_kernel(x_ref, gw_ref, wg_ref, wu_ref, wd_ref, out_ref,
                wgt_ref, wut_ref, wdt_ref, *, tb):
    e = pl.program_id(0)
    f = pl.program_id(1)
    t = pl.program_id(2)

    @pl.when((e == 0) & (f == 0) & (t == 0))
    def _init():
        out_ref[...] = jnp.zeros_like(out_ref)

    @pl.when(t == 0)
    def _prep_weights():
        wgt_ref[...] = wg_ref[0].astype(jnp.bfloat16).T
        wut_ref[...] = wu_ref[0].astype(jnp.bfloat16).T
        wdt_ref[...] = wd_ref[0].astype(jnp.bfloat16).T

    x = x_ref[pl.ds(t * tb, tb), :]
    xg = jax.lax.dot_general(x, wgt_ref[...], (((1,), (0,)), ((), ())),
                             preferred_element_type=jnp.float32)
    xu = jax.lax.dot_general(x, wut_ref[...], (((1,), (0,)), ((), ())),
                             preferred_element_type=jnp.float32)
    h = xg * jax.nn.sigmoid(xg) * xu
    gw = gw_ref[pl.ds(t * tb, tb), :]
    lane = jax.lax.broadcasted_iota(jnp.int32, gw.shape, 1)
    gcol = jnp.sum(jnp.where(lane == e, gw, 0.0), axis=1, keepdims=True)
    hs = (h * gcol).astype(jnp.bfloat16)
    contrib = jax.lax.dot_general(hs, wdt_ref[...], (((1,), (0,)), ((), ())),
                                  preferred_element_type=jnp.float32)
    out_ref[pl.ds(t * tb, tb), :] += contrib


def _pipeline(x, wa2, m, scale, bias, wg_loc, wu_loc, wd_loc, e_off):
    N, D = x.shape
    ER = wa2.shape[0]
    E = m.shape[1]
    E_loc, DFF, _ = wg_loc.shape

    TGB = 512
    gate_out, gw, xb = pl.pallas_call(
        _gate_kernel,
        grid=(N // TGB,),
        in_specs=[
            pl.BlockSpec((TGB, D), lambda t: (t, 0)),
            pl.BlockSpec((ER, D), lambda t: (0, 0)),
            pl.BlockSpec((ER, E), lambda t: (0, 0)),
            pl.BlockSpec((1, E), lambda t: (0, 0)),
            pl.BlockSpec((1, E), lambda t: (0, 0)),
        ],
        out_specs=[
            pl.BlockSpec((TGB, E), lambda t: (t, 0)),
            pl.BlockSpec((TGB, E), lambda t: (t, 0)),
            pl.BlockSpec((TGB, D), lambda t: (t, 0)),
        ],
        out_shape=[
            jax.ShapeDtypeStruct((N, E), jnp.float32),
            jax.ShapeDtypeStruct((N, E), jnp.float32),
            jax.ShapeDtypeStruct((N, D), jnp.bfloat16),
        ],
    )(x, wa2, m, scale, bias)

    gw_loc = jax.lax.dynamic_slice_in_dim(gw, e_off, E_loc, axis=1)

    FB = 768 if DFF % 768 == 0 else DFF
    F = DFF // FB
    TB = 512
    T = N // TB
    out = pl.pallas_call(
        functools.partial(_ffn_kernel, tb=TB),
        grid=(E_loc, F, T),
        in_specs=[
            pl.BlockSpec((N, D), lambda e, f, t: (0, 0)),
            pl.BlockSpec((N, E_loc), lambda e, f, t: (0, 0)),
            pl.BlockSpec((1, FB, D), lambda e, f, t: (e, f, 0)),
            pl.BlockSpec((1, FB, D), lambda e, f, t: (e, f, 0)),
            pl.BlockSpec((1, D, FB), lambda e, f, t: (e, 0, f)),
        ],
        out_specs=pl.BlockSpec((N, D), lambda e, f, t: (0, 0)),
        out_shape=jax.ShapeDtypeStruct((N, D), jnp.float32),
        scratch_shapes=[
            pltpu.VMEM((D, FB), jnp.bfloat16),
            pltpu.VMEM((D, FB), jnp.bfloat16),
            pltpu.VMEM((FB, D), jnp.bfloat16),
        ],
        compiler_params=pltpu.CompilerParams(
            dimension_semantics=("arbitrary", "arbitrary", "arbitrary"),
            vmem_limit_bytes=64 * 1024 * 1024),
    )(xb, gw_loc, wg_loc, wu_loc, wd_loc)
    return out, gate_out


def kernel(hidden_states, W_A, gate_scale, gate_bias, W_gate, W_up, W_down):
    orig_shape = hidden_states.shape
    D = orig_shape[-1]
    x = hidden_states.reshape(-1, D)
    N = x.shape[0]
    E, R, _ = W_A.shape

    wa2 = W_A.reshape(E * R, D)
    m = jnp.repeat(jnp.eye(E, dtype=jnp.float32), R, axis=0) / R
    scale = gate_scale.reshape(1, E)
    bias = gate_bias.reshape(1, E)

    devs = jax.devices()
    ndev = 2 if (len(devs) >= 2 and E % 2 == 0 and N % 2 == 0) else 1

    if ndev == 1:
        out, gate_out = _pipeline(x, wa2, m, scale, bias,
                                  W_gate, W_up, W_down, 0)
        return (out.reshape(orig_shape),
                gate_out.reshape(orig_shape[:-1] + (E,)))

    mesh = Mesh(np.array(devs[:2]), ('d',))
    E_loc = E // 2
    N_loc = N // 2

    def shard_fn(x_s, wa2_s, m_s, scale_s, bias_s, wg_s, wu_s, wd_s):
        idx = jax.lax.axis_index('d')
        out, gate_out = _pipeline(x_s, wa2_s, m_s, scale_s, bias_s,
                                  wg_s, wu_s, wd_s, idx * E_loc)
        out_half = jax.lax.psum_scatter(out, 'd', scatter_dimension=0,
                                        tiled=True)
        gate_half = jax.lax.dynamic_slice_in_dim(gate_out, idx * N_loc,
                                                 N_loc, axis=0)
        return out_half, gate_half

    out, gate_out = jax.shard_map(
        shard_fn,
        mesh=mesh,
        in_specs=(P(), P(), P(), P(), P(),
                  P('d', None, None), P('d', None, None), P('d', None, None)),
        out_specs=(P('d', None), P('d', None)),
        check_vma=False,
    )(x, wa2, m, scale, bias, W_gate, W_up, W_down)

    return out.reshape(orig_shape), gate_out.reshape(orig_shape[:-1] + (E,))


# speedup vs baseline: 1.5709x; 1.5709x over previous
import functools

import jax
import jax.numpy as jnp
import numpy as np
from jax.experimental import pallas as pl
from jax.experimental.pallas import tpu as pltpu
from jax.sharding import Mesh, PartitionSpec as P

_THRESHOLD = 0.5


def _gate_kernel(x_ref, wa_ref, m_ref, scale_ref, bias_ref,
                 gout_ref, gw_ref, xb_ref):
    xb = x_ref[...].astype(jnp.bfloat16)
    xb_ref[...] = xb
    gh = jax.lax.dot_general(
        xb, wa_ref[...].astype(jnp.bfloat16), (((1,), (1,)), ((), ())),
        preferred_element_type=jnp.float32)
    g2 = gh * gh
    s2 = jax.lax.dot_general(
        g2, m_ref[...], (((1,), (0,)), ((), ())),
        precision=jax.lax.Precision.HIGHEST,
        preferred_element_type=jnp.float32)
    scores = jnp.sqrt(s2 + 1e-6) * scale_ref[...] - bias_ref[...]
    mask = scores >= _THRESHOLD
    gout_ref[...] = jnp.where(mask, scores, -jnp.inf)
    gw_ref[...] = jnp.where(mask, scores, 0.0)


def _ffn_kernel(x_ref, gw_ref, wg_ref, wu_ref, wd_ref, out_ref,
                wgt_ref, wut_ref, wdt_ref, *, tb):
    e = pl.program_id(0)
    f = pl.program_id(1)
    t = pl.program_id(2)

    @pl.when((e == 0) & (f == 0) & (t == 0))
    def _init():
        out_ref[...] = jnp.zeros_like(out_ref)

    @pl.when(t == 0)
    def _prep_weights():
        wgt_ref[...] = wg_ref[0].astype(jnp.bfloat16).T
        wut_ref[...] = wu_ref[0].astype(jnp.bfloat16).T
        wdt_ref[...] = wd_ref[0].astype(jnp.bfloat16).T

    x = x_ref[pl.ds(t * tb, tb), :]
    xg = jax.lax.dot_general(x, wgt_ref[...], (((1,), (0,)), ((), ())),
                             preferred_element_type=jnp.float32)
    xu = jax.lax.dot_general(x, wut_ref[...], (((1,), (0,)), ((), ())),
                             preferred_element_type=jnp.float32)
    h = xg * jax.nn.sigmoid(xg) * xu
    gw = gw_ref[pl.ds(t * tb, tb), :]
    lane = jax.lax.broadcasted_iota(jnp.int32, gw.shape, 1)
    gcol = jnp.sum(jnp.where(lane == e, gw, 0.0), axis=1, keepdims=True)
    hs = (h * gcol).astype(jnp.bfloat16)
    contrib = jax.lax.dot_general(hs, wdt_ref[...], (((1,), (0,)), ((), ())),
                                  preferred_element_type=jnp.float32)
    out_ref[pl.ds(t * tb, tb), :] += contrib


def _pipeline(x, wa2, m, scale, bias, wg_loc, wu_loc, wd_loc, e_off):
    N, D = x.shape
    ER = wa2.shape[0]
    E = m.shape[1]
    E_loc, DFF, _ = wg_loc.shape

    TGB = 512
    gate_out, gw, xb = pl.pallas_call(
        _gate_kernel,
        grid=(N // TGB,),
        in_specs=[
            pl.BlockSpec((TGB, D), lambda t: (t, 0)),
            pl.BlockSpec((ER, D), lambda t: (0, 0)),
            pl.BlockSpec((ER, E), lambda t: (0, 0)),
            pl.BlockSpec((1, E), lambda t: (0, 0)),
            pl.BlockSpec((1, E), lambda t: (0, 0)),
        ],
        out_specs=[
            pl.BlockSpec((TGB, E), lambda t: (t, 0)),
            pl.BlockSpec((TGB, E), lambda t: (t, 0)),
            pl.BlockSpec((TGB, D), lambda t: (t, 0)),
        ],
        out_shape=[
            jax.ShapeDtypeStruct((N, E), jnp.float32),
            jax.ShapeDtypeStruct((N, E), jnp.float32),
            jax.ShapeDtypeStruct((N, D), jnp.bfloat16),
        ],
    )(x, wa2, m, scale, bias)

    gw_loc = jax.lax.dynamic_slice_in_dim(gw, e_off, E_loc, axis=1)

    FB = 768 if DFF % 768 == 0 else DFF
    F = DFF // FB
    TB = 1024 if N % 1024 == 0 else N
    T = N // TB
    out = pl.pallas_call(
        functools.partial(_ffn_kernel, tb=TB),
        grid=(E_loc, F, T),
        in_specs=[
            pl.BlockSpec((N, D), lambda e, f, t: (0, 0)),
            pl.BlockSpec((N, E_loc), lambda e, f, t: (0, 0)),
            pl.BlockSpec((1, FB, D), lambda e, f, t: (e, f, 0)),
            pl.BlockSpec((1, FB, D), lambda e, f, t: (e, f, 0)),
            pl.BlockSpec((1, D, FB), lambda e, f, t: (e, 0, f)),
        ],
        out_specs=pl.BlockSpec((N, D), lambda e, f, t: (0, 0)),
        out_shape=jax.ShapeDtypeStruct((N, D), jnp.float32),
        scratch_shapes=[
            pltpu.VMEM((D, FB), jnp.bfloat16),
            pltpu.VMEM((D, FB), jnp.bfloat16),
            pltpu.VMEM((FB, D), jnp.bfloat16),
        ],
        compiler_params=pltpu.CompilerParams(
            dimension_semantics=("arbitrary", "arbitrary", "arbitrary"),
            vmem_limit_bytes=64 * 1024 * 1024),
    )(xb, gw_loc, wg_loc, wu_loc, wd_loc)
    return out, gate_out


def kernel(hidden_states, W_A, gate_scale, gate_bias, W_gate, W_up, W_down):
    orig_shape = hidden_states.shape
    D = orig_shape[-1]
    x = hidden_states.reshape(-1, D)
    N = x.shape[0]
    E, R, _ = W_A.shape

    wa2 = W_A.reshape(E * R, D)
    m = jnp.repeat(jnp.eye(E, dtype=jnp.float32), R, axis=0) / R
    scale = gate_scale.reshape(1, E)
    bias = gate_bias.reshape(1, E)

    out, gate_out = _pipeline(x, wa2, m, scale, bias,
                              W_gate, W_up, W_down, 0)
    return (out.reshape(orig_shape),
            gate_out.reshape(orig_shape[:-1] + (E,)))


# speedup vs baseline: 1.5813x; 1.0066x over previous
import functools

import jax
import jax.numpy as jnp
import numpy as np
from jax.experimental import pallas as pl
from jax.experimental.pallas import tpu as pltpu
from jax.sharding import Mesh, PartitionSpec as P

_THRESHOLD = 0.5


def _gate_kernel(x_ref, wa_ref, m_ref, scale_ref, bias_ref,
                 gout_ref, gw_ref, xb_ref):
    xb = x_ref[...].astype(jnp.bfloat16)
    xb_ref[...] = xb
    gh = jax.lax.dot_general(
        xb, wa_ref[...].astype(jnp.bfloat16), (((1,), (1,)), ((), ())),
        preferred_element_type=jnp.float32)
    g2 = gh * gh
    s2 = jax.lax.dot_general(
        g2, m_ref[...], (((1,), (0,)), ((), ())),
        precision=jax.lax.Precision.HIGHEST,
        preferred_element_type=jnp.float32)
    scores = jnp.sqrt(s2 + 1e-6) * scale_ref[...] - bias_ref[...]
    mask = scores >= _THRESHOLD
    gout_ref[...] = jnp.where(mask, scores, -jnp.inf)
    gw_ref[...] = jnp.where(mask, scores, 0.0)


def _ffn_kernel(x_ref, gw_ref, wg_ref, wu_ref, wd_ref, out_ref,
                wgt_ref, wut_ref, wdt_ref, *, tb):
    e = pl.program_id(0)
    f = pl.program_id(1)
    t = pl.program_id(2)

    @pl.when((e == 0) & (f == 0) & (t == 0))
    def _init():
        out_ref[...] = jnp.zeros_like(out_ref)

    @pl.when(t == 0)
    def _prep_weights():
        wgt_ref[...] = wg_ref[0].astype(jnp.bfloat16)
        wut_ref[...] = wu_ref[0].astype(jnp.bfloat16)
        wdt_ref[...] = wd_ref[0].astype(jnp.bfloat16)

    x = x_ref[pl.ds(t * tb, tb), :]
    xg = jax.lax.dot_general(x, wgt_ref[...], (((1,), (1,)), ((), ())),
                             preferred_element_type=jnp.float32)
    xu = jax.lax.dot_general(x, wut_ref[...], (((1,), (1,)), ((), ())),
                             preferred_element_type=jnp.float32)
    h = xg * jax.nn.sigmoid(xg) * xu
    gw = gw_ref[pl.ds(t * tb, tb), :]
    lane = jax.lax.broadcasted_iota(jnp.int32, gw.shape, 1)
    gcol = jnp.sum(jnp.where(lane == e, gw, 0.0), axis=1, keepdims=True)
    hs = (h * gcol).astype(jnp.bfloat16)
    contrib = jax.lax.dot_general(hs, wdt_ref[...], (((1,), (1,)), ((), ())),
                                  preferred_element_type=jnp.float32)
    out_ref[pl.ds(t * tb, tb), :] += contrib


def _pipeline(x, wa2, m, scale, bias, wg_loc, wu_loc, wd_loc, e_off):
    N, D = x.shape
    ER = wa2.shape[0]
    E = m.shape[1]
    E_loc, DFF, _ = wg_loc.shape

    TGB = 512
    gate_out, gw, xb = pl.pallas_call(
        _gate_kernel,
        grid=(N // TGB,),
        in_specs=[
            pl.BlockSpec((TGB, D), lambda t: (t, 0)),
            pl.BlockSpec((ER, D), lambda t: (0, 0)),
            pl.BlockSpec((ER, E), lambda t: (0, 0)),
            pl.BlockSpec((1, E), lambda t: (0, 0)),
            pl.BlockSpec((1, E), lambda t: (0, 0)),
        ],
        out_specs=[
            pl.BlockSpec((TGB, E), lambda t: (t, 0)),
            pl.BlockSpec((TGB, E), lambda t: (t, 0)),
            pl.BlockSpec((TGB, D), lambda t: (t, 0)),
        ],
        out_shape=[
            jax.ShapeDtypeStruct((N, E), jnp.float32),
            jax.ShapeDtypeStruct((N, E), jnp.float32),
            jax.ShapeDtypeStruct((N, D), jnp.bfloat16),
        ],
    )(x, wa2, m, scale, bias)

    gw_loc = jax.lax.dynamic_slice_in_dim(gw, e_off, E_loc, axis=1)

    FB = 768 if DFF % 768 == 0 else DFF
    F = DFF // FB
    TB = 1024 if N % 1024 == 0 else N
    T = N // TB
    out = pl.pallas_call(
        functools.partial(_ffn_kernel, tb=TB),
        grid=(E_loc, F, T),
        in_specs=[
            pl.BlockSpec((N, D), lambda e, f, t: (0, 0)),
            pl.BlockSpec((N, E_loc), lambda e, f, t: (0, 0)),
            pl.BlockSpec((1, FB, D), lambda e, f, t: (e, f, 0)),
            pl.BlockSpec((1, FB, D), lambda e, f, t: (e, f, 0)),
            pl.BlockSpec((1, D, FB), lambda e, f, t: (e, 0, f)),
        ],
        out_specs=pl.BlockSpec((N, D), lambda e, f, t: (0, 0)),
        out_shape=jax.ShapeDtypeStruct((N, D), jnp.float32),
        scratch_shapes=[
            pltpu.VMEM((FB, D), jnp.bfloat16),
            pltpu.VMEM((FB, D), jnp.bfloat16),
            pltpu.VMEM((D, FB), jnp.bfloat16),
        ],
        compiler_params=pltpu.CompilerParams(
            dimension_semantics=("arbitrary", "arbitrary", "arbitrary"),
            vmem_limit_bytes=64 * 1024 * 1024),
    )(xb, gw_loc, wg_loc, wu_loc, wd_loc)
    return out, gate_out


def kernel(hidden_states, W_A, gate_scale, gate_bias, W_gate, W_up, W_down):
    orig_shape = hidden_states.shape
    D = orig_shape[-1]
    x = hidden_states.reshape(-1, D)
    N = x.shape[0]
    E, R, _ = W_A.shape

    wa2 = W_A.reshape(E * R, D)
    m = jnp.repeat(jnp.eye(E, dtype=jnp.float32), R, axis=0) / R
    scale = gate_scale.reshape(1, E)
    bias = gate_bias.reshape(1, E)

    out, gate_out = _pipeline(x, wa2, m, scale, bias,
                              W_gate, W_up, W_down, 0)
    return (out.reshape(orig_shape),
            gate_out.reshape(orig_shape[:-1] + (E,)))


# speedup vs baseline: 1.6122x; 1.0196x over previous
import functools

import jax
import jax.numpy as jnp
import numpy as np
from jax.experimental import pallas as pl
from jax.experimental.pallas import tpu as pltpu
from jax.sharding import Mesh, PartitionSpec as P

_THRESHOLD = 0.5


def _gate_kernel(x_ref, wa_ref, m_ref, scale_ref, bias_ref,
                 gout_ref, gw_ref, xb_ref):
    xb = x_ref[...].astype(jnp.bfloat16)
    xb_ref[...] = xb
    gh = jax.lax.dot_general(
        xb, wa_ref[...].astype(jnp.bfloat16), (((1,), (1,)), ((), ())),
        preferred_element_type=jnp.float32)
    g2 = gh * gh
    s2 = jax.lax.dot_general(
        g2, m_ref[...], (((1,), (0,)), ((), ())),
        precision=jax.lax.Precision.HIGHEST,
        preferred_element_type=jnp.float32)
    scores = jnp.sqrt(s2 + 1e-6) * scale_ref[...] - bias_ref[...]
    mask = scores >= _THRESHOLD
    gout_ref[...] = jnp.where(mask, scores, -jnp.inf)
    gw_ref[...] = jnp.where(mask, scores, 0.0)


def _ffn_kernel(x_ref, gw_ref, wg_ref, wu_ref, wd_ref, out_ref, *, tb):
    e = pl.program_id(0)
    f = pl.program_id(1)
    t = pl.program_id(2)

    @pl.when((e == 0) & (f == 0) & (t == 0))
    def _init():
        out_ref[...] = jnp.zeros_like(out_ref)

    x = x_ref[pl.ds(t * tb, tb), :]
    wg = wg_ref[0].astype(jnp.bfloat16)
    wu = wu_ref[0].astype(jnp.bfloat16)
    wd = wd_ref[0].astype(jnp.bfloat16)
    xg = jax.lax.dot_general(x, wg, (((1,), (1,)), ((), ())),
                             preferred_element_type=jnp.float32)
    xu = jax.lax.dot_general(x, wu, (((1,), (1,)), ((), ())),
                             preferred_element_type=jnp.float32)
    h = xg * jax.nn.sigmoid(xg) * xu
    gw = gw_ref[pl.ds(t * tb, tb), :]
    lane = jax.lax.broadcasted_iota(jnp.int32, gw.shape, 1)
    gcol = jnp.sum(jnp.where(lane == e, gw, 0.0), axis=1, keepdims=True)
    hs = (h * gcol).astype(jnp.bfloat16)
    contrib = jax.lax.dot_general(hs, wd, (((1,), (1,)), ((), ())),
                                  preferred_element_type=jnp.float32)
    out_ref[pl.ds(t * tb, tb), :] += contrib


def _pipeline(x, wa2, m, scale, bias, wg_loc, wu_loc, wd_loc, e_off):
    N, D = x.shape
    ER = wa2.shape[0]
    E = m.shape[1]
    E_loc, DFF, _ = wg_loc.shape

    TGB = 512
    gate_out, gw, xb = pl.pallas_call(
        _gate_kernel,
        grid=(N // TGB,),
        in_specs=[
            pl.BlockSpec((TGB, D), lambda t: (t, 0)),
            pl.BlockSpec((ER, D), lambda t: (0, 0)),
            pl.BlockSpec((ER, E), lambda t: (0, 0)),
            pl.BlockSpec((1, E), lambda t: (0, 0)),
            pl.BlockSpec((1, E), lambda t: (0, 0)),
        ],
        out_specs=[
            pl.BlockSpec((TGB, E), lambda t: (t, 0)),
            pl.BlockSpec((TGB, E), lambda t: (t, 0)),
            pl.BlockSpec((TGB, D), lambda t: (t, 0)),
        ],
        out_shape=[
            jax.ShapeDtypeStruct((N, E), jnp.float32),
            jax.ShapeDtypeStruct((N, E), jnp.float32),
            jax.ShapeDtypeStruct((N, D), jnp.bfloat16),
        ],
    )(x, wa2, m, scale, bias)

    gw_loc = jax.lax.dynamic_slice_in_dim(gw, e_off, E_loc, axis=1)

    FB = 768 if DFF % 768 == 0 else DFF
    F = DFF // FB
    TB = 1024 if N % 1024 == 0 else N
    T = N // TB
    out = pl.pallas_call(
        functools.partial(_ffn_kernel, tb=TB),
        grid=(E_loc, F, T),
        in_specs=[
            pl.BlockSpec((N, D), lambda e, f, t: (0, 0)),
            pl.BlockSpec((N, E_loc), lambda e, f, t: (0, 0)),
            pl.BlockSpec((1, FB, D), lambda e, f, t: (e, f, 0)),
            pl.BlockSpec((1, FB, D), lambda e, f, t: (e, f, 0)),
            pl.BlockSpec((1, D, FB), lambda e, f, t: (e, 0, f)),
        ],
        out_specs=pl.BlockSpec((N, D), lambda e, f, t: (0, 0)),
        out_shape=jax.ShapeDtypeStruct((N, D), jnp.float32),
        compiler_params=pltpu.CompilerParams(
            dimension_semantics=("arbitrary", "arbitrary", "arbitrary"),
            vmem_limit_bytes=64 * 1024 * 1024),
    )(xb, gw_loc, wg_loc, wu_loc, wd_loc)
    return out, gate_out


def kernel(hidden_states, W_A, gate_scale, gate_bias, W_gate, W_up, W_down):
    orig_shape = hidden_states.shape
    D = orig_shape[-1]
    x = hidden_states.reshape(-1, D)
    N = x.shape[0]
    E, R, _ = W_A.shape

    wa2 = W_A.reshape(E * R, D)
    m = jnp.repeat(jnp.eye(E, dtype=jnp.float32), R, axis=0) / R
    scale = gate_scale.reshape(1, E)
    bias = gate_bias.reshape(1, E)

    out, gate_out = _pipeline(x, wa2, m, scale, bias,
                              W_gate, W_up, W_down, 0)
    return (out.reshape(orig_shape),
            gate_out.reshape(orig_shape[:-1] + (E,)))


# speedup vs baseline: 1.6187x; 1.0040x over previous
import functools

import jax
import jax.numpy as jnp
import numpy as np
from jax.experimental import pallas as pl
from jax.experimental.pallas import tpu as pltpu
from jax.sharding import Mesh, PartitionSpec as P

_THRESHOLD = 0.5


def _gate_kernel(x_ref, wa_ref, m_ref, scale_ref, bias_ref,
                 gout_ref, gw_ref, xb_ref):
    xb = x_ref[...].astype(jnp.bfloat16)
    xb_ref[...] = xb
    gh = jax.lax.dot_general(
        xb, wa_ref[...].astype(jnp.bfloat16), (((1,), (1,)), ((), ())),
        preferred_element_type=jnp.float32)
    g2 = gh * gh
    s2 = jax.lax.dot_general(
        g2, m_ref[...], (((1,), (0,)), ((), ())),
        precision=jax.lax.Precision.HIGHEST,
        preferred_element_type=jnp.float32)
    scores = jnp.sqrt(s2 + 1e-6) * scale_ref[...] - bias_ref[...]
    mask = scores >= _THRESHOLD
    gout_ref[...] = jnp.where(mask, scores, -jnp.inf)
    gw_ref[...] = jnp.where(mask, scores, 0.0)


def _ffn_kernel(x_ref, gw_ref, wg_ref, wu_ref, wd_ref, out_ref, *, tb):
    e = pl.program_id(0)
    f = pl.program_id(1)
    t = pl.program_id(2)

    @pl.when((e == 0) & (f == 0) & (t == 0))
    def _init():
        out_ref[...] = jnp.zeros_like(out_ref)

    x = x_ref[pl.ds(t * tb, tb), :]
    wg = wg_ref[0].astype(jnp.bfloat16)
    wu = wu_ref[0].astype(jnp.bfloat16)
    wd = wd_ref[0].astype(jnp.bfloat16)
    xg = jax.lax.dot_general(x, wg, (((1,), (1,)), ((), ())),
                             preferred_element_type=jnp.float32)
    xu = jax.lax.dot_general(x, wu, (((1,), (1,)), ((), ())),
                             preferred_element_type=jnp.float32)
    h = xg * jax.nn.sigmoid(xg) * xu
    gw = gw_ref[pl.ds(t * tb, tb), :]
    lane = jax.lax.broadcasted_iota(jnp.int32, gw.shape, 1)
    gcol = jnp.sum(jnp.where(lane == e, gw, 0.0), axis=1, keepdims=True)
    hs = (h * gcol).astype(jnp.bfloat16)
    contrib = jax.lax.dot_general(hs, wd, (((1,), (1,)), ((), ())),
                                  preferred_element_type=jnp.float32)
    out_ref[pl.ds(t * tb, tb), :] += contrib


def _pipeline(x, wa2, m, scale, bias, wg_loc, wu_loc, wd_loc, e_off):
    N, D = x.shape
    ER = wa2.shape[0]
    E = m.shape[1]
    E_loc, DFF, _ = wg_loc.shape

    TGB = 512
    gate_out, gw, xb = pl.pallas_call(
        _gate_kernel,
        grid=(N // TGB,),
        in_specs=[
            pl.BlockSpec((TGB, D), lambda t: (t, 0)),
            pl.BlockSpec((ER, D), lambda t: (0, 0)),
            pl.BlockSpec((ER, E), lambda t: (0, 0)),
            pl.BlockSpec((1, E), lambda t: (0, 0)),
            pl.BlockSpec((1, E), lambda t: (0, 0)),
        ],
        out_specs=[
            pl.BlockSpec((TGB, E), lambda t: (t, 0)),
            pl.BlockSpec((TGB, E), lambda t: (t, 0)),
            pl.BlockSpec((TGB, D), lambda t: (t, 0)),
        ],
        out_shape=[
            jax.ShapeDtypeStruct((N, E), jnp.float32),
            jax.ShapeDtypeStruct((N, E), jnp.float32),
            jax.ShapeDtypeStruct((N, D), jnp.bfloat16),
        ],
    )(x, wa2, m, scale, bias)

    gw_loc = jax.lax.dynamic_slice_in_dim(gw, e_off, E_loc, axis=1)

    FB = 768 if DFF % 768 == 0 else DFF
    F = DFF // FB
    TB = 2048 if N % 2048 == 0 else N
    T = N // TB
    out = pl.pallas_call(
        functools.partial(_ffn_kernel, tb=TB),
        grid=(E_loc, F, T),
        in_specs=[
            pl.BlockSpec((N, D), lambda e, f, t: (0, 0)),
            pl.BlockSpec((N, E_loc), lambda e, f, t: (0, 0)),
            pl.BlockSpec((1, FB, D), lambda e, f, t: (e, f, 0)),
            pl.BlockSpec((1, FB, D), lambda e, f, t: (e, f, 0)),
            pl.BlockSpec((1, D, FB), lambda e, f, t: (e, 0, f)),
        ],
        out_specs=pl.BlockSpec((N, D), lambda e, f, t: (0, 0)),
        out_shape=jax.ShapeDtypeStruct((N, D), jnp.float32),
        compiler_params=pltpu.CompilerParams(
            dimension_semantics=("arbitrary", "arbitrary", "arbitrary"),
            vmem_limit_bytes=64 * 1024 * 1024),
    )(xb, gw_loc, wg_loc, wu_loc, wd_loc)
    return out, gate_out


def kernel(hidden_states, W_A, gate_scale, gate_bias, W_gate, W_up, W_down):
    orig_shape = hidden_states.shape
    D = orig_shape[-1]
    x = hidden_states.reshape(-1, D)
    N = x.shape[0]
    E, R, _ = W_A.shape

    wa2 = W_A.reshape(E * R, D)
    m = jnp.repeat(jnp.eye(E, dtype=jnp.float32), R, axis=0) / R
    scale = gate_scale.reshape(1, E)
    bias = gate_bias.reshape(1, E)

    out, gate_out = _pipeline(x, wa2, m, scale, bias,
                              W_gate, W_up, W_down, 0)
    return (out.reshape(orig_shape),
            gate_out.reshape(orig_shape[:-1] + (E,)))
